# baseline (speedup 1.0000x reference)
import jax
import jax.numpy as jnp
from jax.experimental import pallas as pl

E = 8
D = 1024
F = 128
N = 8192
BN = 512


def _fused_kernel(x_ref, wgn_ref, bgn_ref, eps_ref, w_ref, b_ref, out_ref):
    x = x_ref[...]
    gn = jnp.dot(x, wgn_ref[...], preferred_element_type=jnp.float32)
    gn = gn + bgn_ref[...]
    g = gn[:, :E]
    sp_in = gn[:, E:]
    sp = jnp.maximum(sp_in, 0.0) + jnp.log1p(jnp.exp(-jnp.abs(sp_in)))
    h = g + eps_ref[...] * sp
    k = jnp.argmax(h, axis=1).reshape(BN, 1)
    for e in range(E):
        mm = jnp.dot(x, w_ref[e], preferred_element_type=jnp.float32)
        mm = mm + b_ref[e].reshape(1, F)
        sel = (k >= e).astype(jnp.float32)
        out_ref[:, e * F:(e + 1) * F] = mm * sel


def kernel(x, Wg_w, Wg_b, Wn_w, Wn_b, expert_w, expert_b):
    wgn = jnp.concatenate([Wg_w, Wn_w], axis=1)
    bgn = jnp.concatenate([Wg_b, Wn_b]).reshape(1, 2 * E)
    eps = jax.random.normal(jax.random.key(1), (E,), dtype=jnp.float32)
    eps = eps.reshape(1, E)

    grid = (N // BN,)
    return pl.pallas_call(
        _fused_kernel,
        grid=grid,
        in_specs=[
            pl.BlockSpec((BN, D), lambda i: (i, 0)),
            pl.BlockSpec((D, 2 * E), lambda i: (0, 0)),
            pl.BlockSpec((1, 2 * E), lambda i: (0, 0)),
            pl.BlockSpec((1, E), lambda i: (0, 0)),
            pl.BlockSpec((E, D, F), lambda i: (0, 0, 0)),
            pl.BlockSpec((E, F), lambda i: (0, 0)),
        ],
        out_specs=pl.BlockSpec((BN, E * F), lambda i: (i, 0)),
        out_shape=jax.ShapeDtypeStruct((N, E * F), jnp.float32),
    )(x, wgn, bgn, eps, expert_w, expert_b)


# speedup vs baseline: 1.2634x; 1.2634x over previous
import jax
import jax.numpy as jnp
from jax.experimental import pallas as pl
from jax.experimental.pallas import tpu as pltpu

E = 8
D = 1024
F = 128
N = 8192
BN = 512


def _fused_kernel(x_ref, wgn_ref, bgn_ref, eps_ref, w_ref, b_ref, out_ref,
                  wt_ref):
    @pl.when(pl.program_id(0) == 0)
    def _repack():
        for e in range(E):
            wt_ref[:, e * F:(e + 1) * F] = w_ref[e]

    x = x_ref[...]
    gn = jnp.dot(x, wgn_ref[...], preferred_element_type=jnp.float32)
    gn = gn + bgn_ref[...]
    g = gn[:, :E]
    sp_in = gn[:, E:]
    sp = jnp.maximum(sp_in, 0.0) + jnp.log1p(jnp.exp(-jnp.abs(sp_in)))
    h = g + eps_ref[...] * sp
    k = jnp.argmax(h, axis=1).reshape(BN, 1)
    mm = jnp.dot(x, wt_ref[...], preferred_element_type=jnp.float32)
    for e in range(E):
        sel = (k >= e).astype(jnp.float32)
        chunk = (mm[:, e * F:(e + 1) * F] + b_ref[e].reshape(1, F)) * sel
        out_ref[:, e * F:(e + 1) * F] = chunk


def kernel(x, Wg_w, Wg_b, Wn_w, Wn_b, expert_w, expert_b):
    wgn = jnp.concatenate([Wg_w, Wn_w], axis=1)
    bgn = jnp.concatenate([Wg_b, Wn_b]).reshape(1, 2 * E)
    eps = jax.random.normal(jax.random.key(1), (E,), dtype=jnp.float32)
    eps = eps.reshape(1, E)

    grid = (N // BN,)
    return pl.pallas_call(
        _fused_kernel,
        grid=grid,
        in_specs=[
            pl.BlockSpec((BN, D), lambda i: (i, 0)),
            pl.BlockSpec((D, 2 * E), lambda i: (0, 0)),
            pl.BlockSpec((1, 2 * E), lambda i: (0, 0)),
            pl.BlockSpec((1, E), lambda i: (0, 0)),
            pl.BlockSpec((E, D, F), lambda i: (0, 0, 0)),
            pl.BlockSpec((E, F), lambda i: (0, 0)),
        ],
        out_specs=pl.BlockSpec((BN, E * F), lambda i: (i, 0)),
        out_shape=jax.ShapeDtypeStruct((N, E * F), jnp.float32),
        scratch_shapes=[pltpu.VMEM((D, E * F), jnp.float32)],
    )(x, wgn, bgn, eps, expert_w, expert_b)


# speedup vs baseline: 1.3996x; 1.1078x over previous
import jax
import jax.numpy as jnp
from jax.experimental import pallas as pl
from jax.experimental.pallas import tpu as pltpu

E = 8
D = 1024
F = 128
N = 8192
BN = 1024


def _fused_kernel(x_ref, wgn_ref, bgn_ref, eps_ref, w_ref, b_ref, out_ref,
                  wt_ref):
    @pl.when(pl.program_id(0) == 0)
    def _repack():
        for e in range(E):
            wt_ref[:, e * F:(e + 1) * F] = w_ref[e]

    x = x_ref[...]
    gn = jnp.dot(x, wgn_ref[...], preferred_element_type=jnp.float32)
    gn = gn + bgn_ref[...]
    g = gn[:, :E]
    sp_in = gn[:, E:]
    sp = jnp.maximum(sp_in, 0.0) + jnp.log1p(jnp.exp(-jnp.abs(sp_in)))
    h = g + eps_ref[...] * sp
    k = jnp.argmax(h, axis=1).reshape(BN, 1)
    mm = jnp.dot(x, wt_ref[...], preferred_element_type=jnp.float32)
    for e in range(E):
        sel = (k >= e).astype(jnp.float32)
        chunk = (mm[:, e * F:(e + 1) * F] + b_ref[e].reshape(1, F)) * sel
        out_ref[:, e * F:(e + 1) * F] = chunk


def kernel(x, Wg_w, Wg_b, Wn_w, Wn_b, expert_w, expert_b):
    wgn = jnp.concatenate([Wg_w, Wn_w], axis=1)
    bgn = jnp.concatenate([Wg_b, Wn_b]).reshape(1, 2 * E)
    eps = jax.random.normal(jax.random.key(1), (E,), dtype=jnp.float32)
    eps = eps.reshape(1, E)

    grid = (N // BN,)
    return pl.pallas_call(
        _fused_kernel,
        grid=grid,
        in_specs=[
            pl.BlockSpec((BN, D), lambda i: (i, 0)),
            pl.BlockSpec((D, 2 * E), lambda i: (0, 0)),
            pl.BlockSpec((1, 2 * E), lambda i: (0, 0)),
            pl.BlockSpec((1, E), lambda i: (0, 0)),
            pl.BlockSpec((E, D, F), lambda i: (0, 0, 0)),
            pl.BlockSpec((E, F), lambda i: (0, 0)),
        ],
        out_specs=pl.BlockSpec((BN, E * F), lambda i: (i, 0)),
        out_shape=jax.ShapeDtypeStruct((N, E * F), jnp.float32),
        scratch_shapes=[pltpu.VMEM((D, E * F), jnp.float32)],
    )(x, wgn, bgn, eps, expert_w, expert_b)
